# baseline (speedup 1.0000x reference)
import jax
import jax.numpy as jnp
from jax.experimental import pallas as pl
from jax.experimental.pallas import tpu as pltpu

_ROWS = 2 * 8192
_COLS = 2048
_CH_ROWS = 512
_NCH = _ROWS // _CH_ROWS
_NBUF = 3


def _mse_kernel(p_hbm, t_hbm, out_ref, pbuf, tbuf, psem, tsem):
    def p_copy(b, ci):
        return pltpu.make_async_copy(
            p_hbm.at[pl.ds(ci * _CH_ROWS, _CH_ROWS)], pbuf.at[b], psem.at[b])

    def t_copy(b, ci):
        return pltpu.make_async_copy(
            t_hbm.at[pl.ds(ci * _CH_ROWS, _CH_ROWS)], tbuf.at[b], tsem.at[b])

    for ci in range(_NBUF):
        p_copy(ci, ci).start()
        t_copy(ci, ci).start()

    s = 0.0
    c = 0.0
    for ci in range(_NCH):
        b = ci % _NBUF
        p_copy(b, ci).wait()
        t_copy(b, ci).wait()
        p = pbuf[b]
        t = tbuf[b]
        d = p - t
        mask = t != 0.0
        s += jnp.sum(jnp.where(mask, d * d, 0.0))
        c += jnp.sum(jnp.where(mask, 1.0, 0.0))
        nxt = ci + _NBUF
        if nxt < _NCH:
            p_copy(b, nxt).start()
            t_copy(b, nxt).start()

    out_ref[0] = s / c


def kernel(prediction, target):
    p = prediction.reshape(_ROWS, _COLS)
    t = target.reshape(_ROWS, _COLS)
    out = pl.pallas_call(
        _mse_kernel,
        in_specs=[
            pl.BlockSpec(memory_space=pltpu.HBM),
            pl.BlockSpec(memory_space=pltpu.HBM),
        ],
        out_specs=pl.BlockSpec(memory_space=pltpu.SMEM),
        out_shape=jax.ShapeDtypeStruct((1,), jnp.float32),
        scratch_shapes=[
            pltpu.VMEM((_NBUF, _CH_ROWS, _COLS), jnp.float32),
            pltpu.VMEM((_NBUF, _CH_ROWS, _COLS), jnp.float32),
            pltpu.SemaphoreType.DMA((_NBUF,)),
            pltpu.SemaphoreType.DMA((_NBUF,)),
        ],
    )(p, t)
    return out[0]


# speedup vs baseline: 1.0069x; 1.0069x over previous
import jax
import jax.numpy as jnp
from jax.experimental import pallas as pl
from jax.experimental.pallas import tpu as pltpu

_ROWS = 2 * 8192
_COLS = 2048
_CH_ROWS = 1024
_NCH = _ROWS // _CH_ROWS
_NBUF = 3


def _mse_kernel(p_hbm, t_hbm, out_ref, pbuf, tbuf, psem, tsem):
    def p_copy(b, ci):
        return pltpu.make_async_copy(
            p_hbm.at[pl.ds(ci * _CH_ROWS, _CH_ROWS)], pbuf.at[b], psem.at[b])

    def t_copy(b, ci):
        return pltpu.make_async_copy(
            t_hbm.at[pl.ds(ci * _CH_ROWS, _CH_ROWS)], tbuf.at[b], tsem.at[b])

    for ci in range(_NBUF):
        p_copy(ci, ci).start()
        t_copy(ci, ci).start()

    s = 0.0
    c = 0.0
    for ci in range(_NCH):
        b = ci % _NBUF
        p_copy(b, ci).wait()
        t_copy(b, ci).wait()
        p = pbuf[b]
        t = tbuf[b]
        d = p - t
        mask = t != 0.0
        s += jnp.sum(jnp.where(mask, d * d, 0.0))
        c += jnp.sum(jnp.where(mask, 1.0, 0.0))
        nxt = ci + _NBUF
        if nxt < _NCH:
            p_copy(b, nxt).start()
            t_copy(b, nxt).start()

    out_ref[0] = s / c


def kernel(prediction, target):
    p = prediction.reshape(_ROWS, _COLS)
    t = target.reshape(_ROWS, _COLS)
    out = pl.pallas_call(
        _mse_kernel,
        in_specs=[
            pl.BlockSpec(memory_space=pltpu.HBM),
            pl.BlockSpec(memory_space=pltpu.HBM),
        ],
        out_specs=pl.BlockSpec(memory_space=pltpu.SMEM),
        out_shape=jax.ShapeDtypeStruct((1,), jnp.float32),
        scratch_shapes=[
            pltpu.VMEM((_NBUF, _CH_ROWS, _COLS), jnp.float32),
            pltpu.VMEM((_NBUF, _CH_ROWS, _COLS), jnp.float32),
            pltpu.SemaphoreType.DMA((_NBUF,)),
            pltpu.SemaphoreType.DMA((_NBUF,)),
        ],
    )(p, t)
    return out[0]
